# baseline (speedup 1.0000x reference)
import functools

import jax
import jax.numpy as jnp
from jax import lax
from jax.experimental import pallas as pl
from jax.experimental.pallas import tpu as pltpu
from jax.experimental.pallas import tpu_sc as plsc

F = 128
N = 10000
E = 320000
G = 256
N_PAD = 10240
DUMMY = 10000
CHUNK = 120
NC, NS = 2, 16
NW = NC * NS
CW = 84
CW_STRIDE = CW + 2
E_PAD = NW * CW * CHUNK
E_LAYOUT = NW * CW_STRIDE * CHUNK
ROWS_PER_TILE = N_PAD // NS
TRASH = 10112

_f32 = jnp.float32


def _sc_scatter(m_pad, src, dst, ztile):
    out_t = jax.ShapeDtypeStruct((N_PAD, F), _f32)
    mesh = plsc.VectorSubcoreMesh(core_axis_name="c", subcore_axis_name="s")

    idx_t = pltpu.VMEM((CHUNK,), jnp.int32)
    row_t = pltpu.VMEM((CHUNK, F), _f32)

    @functools.partial(
        pl.kernel,
        out_type=(out_t, out_t),
        mesh=mesh,
        scratch_types=[
            idx_t, idx_t, idx_t,
            idx_t, idx_t, idx_t, idx_t,
            idx_t,
            row_t, row_t, row_t,
            pltpu.VMEM_SHARED((N_PAD, F), _f32),
            pltpu.SemaphoreType.DMA,
            pltpu.SemaphoreType.DMA,
            pltpu.SemaphoreType.DMA,
            pltpu.SemaphoreType.DMA,
            pltpu.SemaphoreType.DMA,
        ],
    )
    def k(m_hbm, src_hbm, dst_hbm, z_hbm, out0, out1,
          s0, s1, s2, d0, d1, d2, d3, dP, r0, r1, r2, acc,
          semG, semS0, semS1, semS2, semI):
        S = (s0, s1, s2)
        D = (d0, d1, d2, d3)
        R = (r0, r1, r2)
        semS = (semS0, semS1, semS2)
        core = lax.axis_index("c")
        sid = lax.axis_index("s")
        w = core * NS + sid
        my_rows = pl.ds(sid * ROWS_PER_TILE, ROWS_PER_TILE)
        eb = w * CW_STRIDE * CHUNK

        pltpu.sync_copy(src_hbm.at[pl.ds(eb, CHUNK)], S[0])
        pltpu.sync_copy(dst_hbm.at[pl.ds(eb, CHUNK)], D[0])
        pltpu.async_copy(src_hbm.at[pl.ds(eb + CHUNK, CHUNK)], S[1], semI)
        pltpu.async_copy(dst_hbm.at[pl.ds(eb + CHUNK, CHUNK)], D[1], semI)
        pltpu.sync_copy(dst_hbm.at[pl.ds(eb + (CW + 1) * CHUNK, CHUNK)], dP)
        pltpu.sync_copy(z_hbm, acc.at[my_rows])
        plsc.subcore_barrier()
        pltpu.async_copy(m_hbm.at[S[0]], R[0], semG)
        pltpu.async_copy(R[1], acc.at[dP], semS[1], add=True)
        pltpu.async_copy(R[2], acc.at[dP], semS[2], add=True)

        def half(q, k_):
            r, r1, r2 = k_ % 3, (k_ + 1) % 3, (k_ + 2) % 3
            dq, dq2 = k_ % 4, (k_ + 2) % 4
            pltpu.make_async_copy(R[r1], acc.at[D[dq]], semS[r1]).wait()
            pltpu.make_async_copy(src_hbm.at[pl.ds(eb, CHUNK)], S[r1], semI).wait()
            pltpu.make_async_copy(dst_hbm.at[pl.ds(eb, CHUNK)], D[dq], semI).wait()
            nxt = eb + (q + 2) * CHUNK
            pltpu.async_copy(src_hbm.at[pl.ds(nxt, CHUNK)], S[r2], semI)
            pltpu.async_copy(dst_hbm.at[pl.ds(nxt, CHUNK)], D[dq2], semI)
            pltpu.make_async_copy(m_hbm.at[S[r]], R[r], semG).wait()
            pltpu.async_copy(R[r], acc.at[D[dq]], semS[r], add=True)
            pltpu.async_copy(m_hbm.at[S[r1]], R[r1], semG)

        @pl.loop(0, CW, step=12)
        def _(j):
            for k_ in range(12):
                half(j + k_, k_)

        pltpu.make_async_copy(m_hbm.at[S[0]], R[0], semG).wait()
        pltpu.make_async_copy(R[1], acc.at[D[0]], semS[1]).wait()
        pltpu.make_async_copy(R[2], acc.at[D[0]], semS[2]).wait()
        pltpu.make_async_copy(src_hbm.at[pl.ds(eb, CHUNK)], S[0], semI).wait()
        pltpu.make_async_copy(dst_hbm.at[pl.ds(eb, CHUNK)], D[0], semI).wait()

        plsc.subcore_barrier()

        @pl.when(core == 0)
        def _():
            pltpu.sync_copy(acc.at[my_rows], out0.at[my_rows])

        @pl.when(core == 1)
        def _():
            pltpu.sync_copy(acc.at[my_rows], out1.at[my_rows])

    return k(m_pad, src, dst, ztile)


def _mm_body(x_ref, w_ref, o_ref):
    o_ref[...] = jnp.dot(x_ref[...], w_ref[...], preferred_element_type=_f32)


def _mm(x, w):
    return pl.pallas_call(
        _mm_body,
        out_shape=jax.ShapeDtypeStruct((x.shape[0], w.shape[1]), _f32),
    )(x, w)


def _fused_body(a0_ref, a1_ref, m_ref, w_ref, o_ref):
    h = jnp.maximum(a0_ref[...] + a1_ref[...] + m_ref[...], 0.0)
    o_ref[...] = jnp.dot(h, w_ref[...], preferred_element_type=_f32)


def _fused(a0, a1, m, w):
    return pl.pallas_call(
        _fused_body,
        out_shape=jax.ShapeDtypeStruct((m.shape[0], w.shape[1]), _f32),
    )(a0, a1, m, w)


def _pool_body(a0_ref, a1_ref, m_ref, b_ref, o_ref):
    h = jnp.maximum(a0_ref[...] + a1_ref[...] + m_ref[...], 0.0)
    rmask = lax.broadcasted_iota(jnp.int32, (N_PAD, 1), 0) < TRASH
    h = jnp.where(rmask, h, 0.0)
    seg = b_ref[...]
    gi = lax.broadcasted_iota(jnp.int32, (G, N_PAD), 0)
    p = (gi == seg).astype(_f32)
    sums = jnp.dot(p, h, preferred_element_type=_f32)
    counts = jnp.sum(p, axis=1, keepdims=True)
    o_ref[...] = sums / jnp.maximum(counts, 1.0)


def _pool(a0, a1, m, batch2d):
    return pl.pallas_call(
        _pool_body,
        out_shape=jax.ShapeDtypeStruct((G, F), _f32),
    )(a0, a1, m, batch2d)


def _cls_body(h1_ref, h2_ref, w1_ref, b1_ref, w2_ref, b2_ref, w3_ref, b3_ref, o_ref):
    hp = jnp.concatenate([h1_ref[...], h2_ref[...]], axis=1)
    z = jnp.maximum(jnp.dot(hp, w1_ref[...], preferred_element_type=_f32) + b1_ref[...], 0.0)
    z = jnp.maximum(jnp.dot(z, w2_ref[...], preferred_element_type=_f32) + b2_ref[...], 0.0)
    o_ref[...] = jnp.dot(z, w3_ref[...], preferred_element_type=_f32) + b3_ref[...]


def _classifier(h1m, h2m, w1, b1, w2, b2, w3, b3):
    return pl.pallas_call(
        _cls_body,
        out_shape=jax.ShapeDtypeStruct((G, w3.shape[1]), _f32),
    )(h1m, h2m, w1, b1.reshape(1, -1), w2, b2.reshape(1, -1), w3, b3.reshape(1, -1))


def _prep_edges(edge_index):
    e = edge_index.astype(jnp.int32)
    pad = E_PAD - e.shape[1]

    def lay_out(real, fill_base, fill_mod, over85_base, over85_mod):
        f1 = jnp.arange(pad, dtype=jnp.int32)
        v = jnp.concatenate([real, fill_base + f1 % fill_mod])
        v = v.reshape(NW, CW * CHUNK)
        f2 = jnp.arange(NW * CHUNK, dtype=jnp.int32)
        ex84 = (fill_base + f2 % fill_mod).reshape(NW, CHUNK)
        ex85 = (over85_base + f2 % over85_mod).reshape(NW, CHUNK)
        return jnp.concatenate([v, ex84, ex85], axis=1).reshape(E_LAYOUT)

    src = lay_out(e[0], DUMMY, TRASH - DUMMY, DUMMY, TRASH - DUMMY)
    dst = lay_out(e[1], 0, N_PAD, TRASH, N_PAD - TRASH)
    return src, dst


def kernel(drug1_x, drug1_edge_index, drug1_batch,
           drug2_x, drug2_edge_index, drug2_batch,
           enc_W0, enc_W1, enc_W2,
           cls_W1, cls_b1, cls_W2, cls_b2, cls_W3, cls_b3):
    ztile = jnp.zeros((ROWS_PER_TILE, F), _f32)
    Ws = (enc_W0, enc_W1, enc_W2)

    def encode(x, edge_index, batch):
        x_pad = jnp.pad(x, ((0, N_PAD - N), (0, 0)))
        src, dst = _prep_edges(edge_index)
        batch2d = jnp.pad(batch.astype(jnp.int32), (0, N_PAD - N),
                          constant_values=300).reshape(1, N_PAD)
        m = _mm(x_pad, Ws[0])
        for li in (1, 2):
            a0, a1 = _sc_scatter(m, src, dst, ztile)
            m = _fused(a0, a1, m, Ws[li])
        a0, a1 = _sc_scatter(m, src, dst, ztile)
        return _pool(a0, a1, m, batch2d)

    h1m = encode(drug1_x, drug1_edge_index, drug1_batch)
    h2m = encode(drug2_x, drug2_edge_index, drug2_batch)
    return _classifier(h1m, h2m, cls_W1, cls_b1, cls_W2, cls_b2, cls_W3, cls_b3)


# speedup vs baseline: 1.0693x; 1.0693x over previous
import functools

import jax
import jax.numpy as jnp
from jax import lax
from jax.experimental import pallas as pl
from jax.experimental.pallas import tpu as pltpu
from jax.experimental.pallas import tpu_sc as plsc

F = 128
N = 10000
E = 320000
G = 256
N_PAD = 10240
DUMMY = 10000
CHUNK = 128
NC, NS = 2, 16
NW = NC * NS
CW = 80
CW_STRIDE = CW + 2
E_PAD = NW * CW * CHUNK
E_LAYOUT = NW * CW_STRIDE * CHUNK
ROWS_PER_TILE = N_PAD // NS

_f32 = jnp.float32


def _sc_scatter(m_pad, src, dst, ztile):
    out_t = jax.ShapeDtypeStruct((N_PAD, F), _f32)
    mesh = plsc.VectorSubcoreMesh(core_axis_name="c", subcore_axis_name="s")

    @functools.partial(
        pl.kernel,
        out_type=(out_t, out_t),
        mesh=mesh,
        scratch_types=[
            pltpu.VMEM((CHUNK,), jnp.int32),
            pltpu.VMEM((CHUNK,), jnp.int32),
            pltpu.VMEM((CHUNK,), jnp.int32),
            pltpu.VMEM((CHUNK,), jnp.int32),
            pltpu.VMEM((CHUNK, F), _f32),
            pltpu.VMEM((CHUNK, F), _f32),
            pltpu.VMEM_SHARED((N_PAD, F), _f32),
            pltpu.SemaphoreType.DMA,
            pltpu.SemaphoreType.DMA,
            pltpu.SemaphoreType.DMA,
        ],
    )
    def k(m_hbm, src_hbm, dst_hbm, z_hbm, out0, out1,
          sidx0, didx0, sidx1, didx1, rows0, rows1, acc, semG0, semG1, semI):
        core = lax.axis_index("c")
        sid = lax.axis_index("s")
        w = core * NS + sid
        my_rows = pl.ds(sid * ROWS_PER_TILE, ROWS_PER_TILE)
        eb = w * CW_STRIDE * CHUNK

        pltpu.sync_copy(src_hbm.at[pl.ds(eb, CHUNK)], sidx0)
        pltpu.sync_copy(dst_hbm.at[pl.ds(eb, CHUNK)], didx0)
        pltpu.async_copy(src_hbm.at[pl.ds(eb + CHUNK, CHUNK)], sidx1, semI)
        pltpu.async_copy(dst_hbm.at[pl.ds(eb + CHUNK, CHUNK)], didx1, semI)
        pltpu.sync_copy(z_hbm.at[my_rows], acc.at[my_rows])
        plsc.subcore_barrier()
        pltpu.async_copy(m_hbm.at[sidx0], rows0, semG0)

        def half(q, sA, dA, rA, gA, sB, dB, rB, gB):
            nxt1 = eb + (q + 1) * CHUNK
            pltpu.make_async_copy(src_hbm.at[pl.ds(nxt1, CHUNK)], sB, semI).wait()
            pltpu.make_async_copy(dst_hbm.at[pl.ds(nxt1, CHUNK)], dB, semI).wait()
            pltpu.async_copy(m_hbm.at[sB], rB, gB)
            pltpu.make_async_copy(m_hbm.at[sA], rA, gA).wait()
            pltpu.sync_copy(rA, acc.at[dA], add=True)
            nxt2 = eb + (q + 2) * CHUNK
            pltpu.async_copy(src_hbm.at[pl.ds(nxt2, CHUNK)], sA, semI)
            pltpu.async_copy(dst_hbm.at[pl.ds(nxt2, CHUNK)], dA, semI)

        @pl.loop(0, CW, step=2)
        def _(j):
            half(j, sidx0, didx0, rows0, semG0, sidx1, didx1, rows1, semG1)
            half(j + 1, sidx1, didx1, rows1, semG1, sidx0, didx0, rows0, semG0)

        pltpu.make_async_copy(m_hbm.at[sidx0], rows0, semG0).wait()
        pltpu.make_async_copy(src_hbm.at[pl.ds(eb, CHUNK)], sidx1, semI).wait()
        pltpu.make_async_copy(dst_hbm.at[pl.ds(eb, CHUNK)], didx1, semI).wait()

        plsc.subcore_barrier()

        @pl.when(core == 0)
        def _():
            pltpu.sync_copy(acc.at[my_rows], out0.at[my_rows])

        @pl.when(core == 1)
        def _():
            pltpu.sync_copy(acc.at[my_rows], out1.at[my_rows])

    return k(m_pad, src, dst, ztile)


def _mm_body(x_ref, w_ref, o_ref):
    o_ref[...] = jnp.dot(x_ref[...], w_ref[...], preferred_element_type=_f32)


def _mm(x, w):
    return pl.pallas_call(
        _mm_body,
        out_shape=jax.ShapeDtypeStruct((x.shape[0], w.shape[1]), _f32),
    )(x, w)


def _fused_body(a0_ref, a1_ref, m_ref, w_ref, o_ref):
    h = jnp.maximum(a0_ref[...] + a1_ref[...] + m_ref[...], 0.0)
    o_ref[...] = jnp.dot(h, w_ref[...], preferred_element_type=_f32)


def _fused(a0, a1, m, w):
    return pl.pallas_call(
        _fused_body,
        out_shape=jax.ShapeDtypeStruct((m.shape[0], w.shape[1]), _f32),
    )(a0, a1, m, w)


def _pool_body(a0_ref, a1_ref, m_ref, b_ref, o_ref):
    h = jnp.maximum(a0_ref[...] + a1_ref[...] + m_ref[...], 0.0)
    seg = b_ref[...]
    gi = lax.broadcasted_iota(jnp.int32, (G, N_PAD), 0)
    p = (gi == seg).astype(_f32)
    sums = jnp.dot(p, h, preferred_element_type=_f32)
    counts = jnp.sum(p, axis=1, keepdims=True)
    o_ref[...] = sums / jnp.maximum(counts, 1.0)


def _pool(a0, a1, m, batch2d):
    return pl.pallas_call(
        _pool_body,
        out_shape=jax.ShapeDtypeStruct((G, F), _f32),
    )(a0, a1, m, batch2d)


def _cls_body(h1_ref, h2_ref, w1_ref, b1_ref, w2_ref, b2_ref, w3_ref, b3_ref, o_ref):
    hp = jnp.concatenate([h1_ref[...], h2_ref[...]], axis=1)
    z = jnp.maximum(jnp.dot(hp, w1_ref[...], preferred_element_type=_f32) + b1_ref[...], 0.0)
    z = jnp.maximum(jnp.dot(z, w2_ref[...], preferred_element_type=_f32) + b2_ref[...], 0.0)
    o_ref[...] = jnp.dot(z, w3_ref[...], preferred_element_type=_f32) + b3_ref[...]


def _classifier(h1m, h2m, w1, b1, w2, b2, w3, b3):
    return pl.pallas_call(
        _cls_body,
        out_shape=jax.ShapeDtypeStruct((G, w3.shape[1]), _f32),
    )(h1m, h2m, w1, b1.reshape(1, -1), w2, b2.reshape(1, -1), w3, b3.reshape(1, -1))


def _prep_edges(edge_index):
    e = edge_index.astype(jnp.int32)
    pad = E_PAD - e.shape[1]

    def lay_out(real, fill_base, fill_mod):
        f1 = jnp.arange(pad, dtype=jnp.int32)
        v = jnp.concatenate([real, fill_base + f1 % fill_mod])
        v = v.reshape(NW, CW * CHUNK)
        f2 = jnp.arange(NW * (CW_STRIDE - CW) * CHUNK, dtype=jnp.int32)
        ex = (fill_base + f2 % fill_mod).reshape(NW, (CW_STRIDE - CW) * CHUNK)
        return jnp.concatenate([v, ex], axis=1).reshape(E_LAYOUT)

    src = lay_out(e[0], DUMMY, N_PAD - DUMMY)
    dst = lay_out(e[1], 0, N_PAD)
    return src, dst


def kernel(drug1_x, drug1_edge_index, drug1_batch,
           drug2_x, drug2_edge_index, drug2_batch,
           enc_W0, enc_W1, enc_W2,
           cls_W1, cls_b1, cls_W2, cls_b2, cls_W3, cls_b3):
    ztile = jnp.zeros((N_PAD, F), _f32)
    Ws = (enc_W0, enc_W1, enc_W2)

    def encode(x, edge_index, batch):
        x_pad = jnp.pad(x, ((0, N_PAD - N), (0, 0)))
        src, dst = _prep_edges(edge_index)
        batch2d = jnp.pad(batch.astype(jnp.int32), (0, N_PAD - N),
                          constant_values=300).reshape(1, N_PAD)
        m = _mm(x_pad, Ws[0])
        for li in (1, 2):
            a0, a1 = _sc_scatter(m, src, dst, ztile)
            m = _fused(a0, a1, m, Ws[li])
        a0, a1 = _sc_scatter(m, src, dst, ztile)
        return _pool(a0, a1, m, batch2d)

    h1m = encode(drug1_x, drug1_edge_index, drug1_batch)
    h2m = encode(drug2_x, drug2_edge_index, drug2_batch)
    return _classifier(h1m, h2m, cls_W1, cls_b1, cls_W2, cls_b2, cls_W3, cls_b3)
